# baseline (speedup 1.0000x reference)
import jax
import jax.numpy as jnp
from jax.experimental import pallas as pl
from jax.experimental.pallas import tpu as pltpu

_BLOCK_ROWS = 4096
_COL_CHUNKS = 3


def _router_body(*refs):
    x_refs = refs[:_COL_CHUNKS]
    w_refs = refs[_COL_CHUNKS:2 * _COL_CHUNKS]
    b_ref = refs[2 * _COL_CHUNKS]
    out_ref = refs[2 * _COL_CHUNKS + 1]
    logits = b_ref[...][None, :]
    for xr, wr in zip(x_refs, w_refs):
        logits = logits + jax.lax.dot_general(
            xr[...], wr[...], (((1,), (0,)), ((), ())),
            preferred_element_type=jnp.float32,
        )
    m = jnp.max(logits, axis=-1, keepdims=True)
    e = jnp.exp(logits - m)
    out_ref[...] = e / jnp.sum(e, axis=-1, keepdims=True)


def kernel(x_t, mem, W, b):
    n, d = x_t.shape
    n_exp = W.shape[1]
    dc = d // _COL_CHUNKS
    grid = (n // _BLOCK_ROWS,)

    def x_spec(k):
        return pl.BlockSpec((_BLOCK_ROWS, dc), lambda i, k=k: (i, k))

    def w_spec(k):
        return pl.BlockSpec((dc, n_exp), lambda i, k=k: (k, 0))

    probs = pl.pallas_call(
        _router_body,
        grid=grid,
        in_specs=(
            [x_spec(k) for k in range(_COL_CHUNKS)]
            + [w_spec(k) for k in range(_COL_CHUNKS)]
            + [pl.BlockSpec((n_exp,), lambda i: (0,))]
        ),
        out_specs=pl.BlockSpec((_BLOCK_ROWS, n_exp), lambda i: (i, 0)),
        out_shape=jax.ShapeDtypeStruct((n, n_exp), jnp.float32),
    )(*([x_t] * _COL_CHUNKS + [W] * _COL_CHUNKS + [b]))
    return (probs, mem)
